# baseline (speedup 1.0000x reference)
import functools

import jax
import jax.numpy as jnp
from jax import lax
from jax.experimental import pallas as pl
from jax.experimental.pallas import tpu as pltpu
from jax.experimental.pallas import tpu_sc as plsc

N_EDGES = 320000
N_SEG = 10000
D = 128
NB = D // 16

NC = 2
NS = 16
NW = NC * NS
E_PER_TILE = N_EDGES // NW
CH = 80
NWIN = CH // 16
NCH = E_PER_TILE // CH
PART_CAP = 80
FLUSH_AT = PART_CAP - 17
POOL_ROWS = N_SEG + 16
EXP_STRIDE = 624
EXP_ROWS = 640


def _sc_body(x, idsr, zeros, lane, out, pooled, idxb, part_val, part_idx,
             pib, lanebuf, vb, ldsem):
    c = lax.axis_index("c")
    s = lax.axis_index("s")
    wid = s * NC + c
    ebase = wid * E_PER_TILE
    dummy = jnp.broadcast_to(N_SEG + lax.rem(wid, 16), (16,))

    pltpu.sync_copy(zeros, pooled.at[pl.ds(s * EXP_STRIDE, EXP_ROWS)])
    pltpu.sync_copy(idsr.at[wid], idxb)
    pltpu.sync_copy(lane, lanebuf)
    for t in range(PART_CAP // 16):
        pib[t, pl.ds(0, 16)] = dummy
    plsc.subcore_barrier()

    def load_start(j, slot):
        pltpu.async_copy(
            x.at[pl.ds(ebase + j * CH, CH)], vb.at[pl.ds(slot * CH, CH)],
            ldsem.at[slot])

    def load_wait(j, slot):
        pltpu.make_async_copy(
            x.at[pl.ds(ebase + j * CH, CH)], vb.at[pl.ds(slot * CH, CH)],
            ldsem.at[slot]).wait()

    def flush():
        for t in range(PART_CAP // 16):
            part_idx[0, pl.ds(t * 16, 16)] = pib[t, pl.ds(0, 16)]
        pltpu.sync_copy(
            part_val.at[pl.ds(0, PART_CAP)],
            pooled.at[part_idx.at[0]], add=True)
        for t in range(PART_CAP // 16):
            pib[t, pl.ds(0, 16)] = dummy

    load_start(0, 0)

    def step(j, carry):
        p, acc = carry
        slot = lax.rem(j, 2)
        nslot = lax.rem(j + 1, 2)

        @pl.when(j + 1 < NCH)
        def _prefetch():
            load_start(j + 1, nslot)

        load_wait(j, slot)

        for w in range(NWIN):
            @pl.when(p >= FLUSH_AT)
            def _fl():
                flush()

            p = jnp.where(p >= FLUSH_AT, 0, p)
            idv = idxb[j, pl.ds(w * 16, 16)]
            if w < NWIN - 1:
                idn = idxb[j, pl.ds((w + 1) * 16, 16)]
            else:
                idn = idxb[j + 1, pl.ds(0, 16)]
            for r in range(16):
                base = slot * CH + w * 16 + r
                acc = [a + vb[base, pl.ds(k * 16, 16)]
                       for k, a in enumerate(acc)]
                for k in range(NB):
                    part_val[p, pl.ds(k * 16, 16)] = acc[k]
                id_r = idv[r]
                id_r1 = idn[0] if r == 15 else idv[r + 1]
                b = jnp.where(id_r != id_r1, jnp.int32(1), jnp.int32(0))

                @pl.when(b == 1)
                def _close():
                    blk = lax.shift_right_logical(p, 4)
                    ln = lax.bitwise_and(p, 15)
                    oh = jnp.where(
                        lanebuf[pl.ds(0, 16)]
                        == jnp.broadcast_to(ln, (16,)),
                        jnp.full((16,), 1, jnp.int32),
                        jnp.zeros((16,), jnp.int32))
                    cur = pib[blk, pl.ds(0, 16)]
                    pib[blk, pl.ds(0, 16)] = (
                        cur * (1 - oh) + jnp.broadcast_to(id_r, (16,)) * oh)

                keep = jnp.broadcast_to((1 - b).astype(jnp.float32), (16,))
                acc = [a * keep for a in acc]
                p = p + b
        return p, acc

    zacc = [jnp.zeros((16,), jnp.float32) for _ in range(NB)]
    lax.fori_loop(0, NCH, step, (jnp.int32(0), zacc))
    flush()
    plsc.subcore_barrier()

    sl = pl.ds(s * EXP_STRIDE, EXP_ROWS)
    pltpu.sync_copy(pooled.at[sl], out.at[c, sl])


_sc_segsum = functools.partial(
    pl.kernel,
    out_type=jax.ShapeDtypeStruct((NC, N_SEG, D), jnp.float32),
    mesh=plsc.VectorSubcoreMesh(
        core_axis_name="c", subcore_axis_name="s", num_cores=NC,
        num_subcores=NS),
    scratch_types=[
        pltpu.VMEM_SHARED((POOL_ROWS, D), jnp.float32),
        pltpu.VMEM((NCH + 1, CH), jnp.int32),
        pltpu.VMEM((PART_CAP, D), jnp.float32),
        pltpu.VMEM((1, PART_CAP), jnp.int32),
        pltpu.VMEM((PART_CAP // 16, 16), jnp.int32),
        pltpu.VMEM((16,), jnp.int32),
        pltpu.VMEM((2 * CH, D), jnp.float32),
        pltpu.SemaphoreType.DMA((2,)),
    ],
)(_sc_body)


def _tc_body(p_ref, wt_ref, b_ref, o_ref):
    p = p_ref[0] + p_ref[1]
    o_ref[...] = (
        jnp.dot(p, wt_ref[...], preferred_element_type=jnp.float32)
        + b_ref[...]
    )


def _tc_linear(partials, wt, b2):
    return pl.pallas_call(
        _tc_body,
        out_shape=jax.ShapeDtypeStruct((N_SEG, D), jnp.float32),
    )(partials, wt, b2)


def kernel(x, batch, W, b):
    b32 = batch.astype(jnp.int32)
    ids = b32.reshape(NW, NCH, CH)
    sent = jnp.full((NW, 1, CH), -1, jnp.int32)
    ids = jnp.concatenate([ids, sent], axis=1)
    zeros = jnp.zeros((EXP_ROWS, D), jnp.float32)
    lane = jnp.arange(16, dtype=jnp.int32)
    partials = _sc_segsum(x, ids, zeros, lane)
    return _tc_linear(partials, W.T, b.reshape(1, D))
